# baseline (speedup 1.0000x reference)
import math

import jax
import jax.numpy as jnp
from jax.experimental import pallas as pl


def _fma_kernel(x_ref, p_ref, o_ref, *, scale):
    o_ref[...] = x_ref[...] * scale + p_ref[...]


def kernel(x, pos_emb):
    B, L, D = x.shape
    scale = math.sqrt(D)
    LD = L * D
    xf = x.reshape(B, LD)
    pf = pos_emb.reshape(1, LD)
    blk = 128
    import functools
    out = pl.pallas_call(
        functools.partial(_fma_kernel, scale=scale),
        grid=(B // blk,),
        in_specs=[
            pl.BlockSpec((blk, LD), lambda i: (i, 0)),
            pl.BlockSpec((1, LD), lambda i: (0, 0)),
        ],
        out_specs=pl.BlockSpec((blk, LD), lambda i: (i, 0)),
        out_shape=jax.ShapeDtypeStruct((B, LD), x.dtype),
    )(xf, pf)
    return out.reshape(B, L, D)


# speedup vs baseline: 3.5229x; 3.5229x over previous
import functools
import math

import jax
import jax.numpy as jnp
from jax.experimental import pallas as pl


def _fma_kernel(x_ref, p_ref, o_ref, *, scale):
    o_ref[...] = x_ref[...] * scale + p_ref[...]


def kernel(x, pos_emb):
    B, L, D = x.shape
    scale = math.sqrt(D)
    LD = L * D
    xt = x.transpose(1, 2, 0).reshape(LD, B)
    pf = pos_emb.reshape(LD, 1)
    blk = 512
    out = pl.pallas_call(
        functools.partial(_fma_kernel, scale=scale),
        grid=(LD // blk,),
        in_specs=[
            pl.BlockSpec((blk, B), lambda i: (i, 0)),
            pl.BlockSpec((blk, 1), lambda i: (i, 0)),
        ],
        out_specs=pl.BlockSpec((blk, B), lambda i: (i, 0)),
        out_shape=jax.ShapeDtypeStruct((LD, B), x.dtype),
    )(xt, pf)
    return out.reshape(L, D, B).transpose(2, 0, 1)
